# baseline (speedup 1.0000x reference)
import jax
import jax.numpy as jnp
from jax import lax
from jax.experimental import pallas as pl
from jax.experimental.pallas import tpu as pltpu
from jax.experimental.pallas import tpu_sc as plsc

N = 4096
NC = 2
NS = 16
NW = NC * NS
ROWS_PER_W = N // NW
C = 8
NCHUNK = ROWS_PER_W // C
L = 16


def _body(x_hbm, eps_hbm, out_hbm, buf0, buf1, buf2, epsv, xrow,
          sem0, sem1, sem2, esem):
    wid = lax.axis_index("s") * NC + lax.axis_index("c")
    base = wid * ROWS_PER_W

    src_off = jnp.where(wid == 0, 0, base - C)
    dst_off = jnp.where(wid == 0, C, 0)
    eps_cp = pltpu.make_async_copy(
        eps_hbm.at[pl.ds(src_off, ROWS_PER_W + L - C)],
        epsv.at[pl.ds(dst_off, ROWS_PER_W + L - C)],
        esem,
    )
    eps_cp.start()

    zeros = jnp.zeros((L,), jnp.float32)
    izeros = jnp.zeros((L,), jnp.int32)

    def _zero_buf(buf):
        for j in range(C):
            def trip(i, _):
                b0 = i * (8 * L)
                for u in range(8):
                    buf[j, pl.ds(b0 + u * L, L)] = zeros
                return 0
            lax.fori_loop(0, N // (8 * L), trip, 0)

    bufs = (buf0, buf1, buf2)
    sems = (sem0, sem1, sem2)
    NB = len(bufs)
    pending = [None, None, None]

    lane = lax.iota(jnp.int32, L)
    lane_ok = (lane >= C - 1) & (lane < L - 1)

    def diag_pos(k):
        col = base - C + 8 * k + lane
        m = lane_ok & (col >= 0)
        return m, jnp.where(m, lane - (C - 1), 0), jnp.where(m, col, 0)

    def run_chunk(k):
        b = k % NB
        if pending[b] is not None:
            pending[b].wait()
            om, orow, ocol = diag_pos(k - NB)
            plsc.store_scatter(bufs[b], [orow, ocol], zeros, mask=om)
        m, row, col = diag_pos(k)
        ev = epsv[pl.ds(8 * k, L)]
        plsc.store_scatter(bufs[b], [row, col], ev, mask=m)
        pending[b] = pltpu.async_copy(
            bufs[b],
            out_hbm.at[pl.ds(base + 8 * k, C)],
            sems[b],
        )

    _zero_buf(buf0)
    eps_cp.wait()
    run_chunk(0)
    _zero_buf(buf1)
    run_chunk(1)
    _zero_buf(buf2)
    for k in range(2, NCHUNK):
        run_chunk(k)
    for p in pending:
        p.wait()

    @pl.when(wid == 0)
    def _():
        pltpu.sync_copy(x_hbm, xrow)
        pltpu.sync_copy(xrow, out_hbm.at[pl.ds(0, 1)])

    @pl.when(wid == NW - 1)
    def _():
        last0 = NB * ((NCHUNK - 1) // NB)
        om, orow, ocol = diag_pos(last0)
        plsc.store_scatter(buf0, [orow, ocol], zeros, mask=om)
        one = lane == C - 1
        ev = epsv[pl.ds(8 * NCHUNK, L)]
        plsc.store_scatter(buf0, [izeros, jnp.where(one, N - 1, 0)],
                           ev, mask=one)
        pltpu.sync_copy(buf0.at[pl.ds(0, 1)], out_hbm.at[pl.ds(N, 1)])


@jax.jit
def kernel(x, eps):
    mesh = plsc.VectorSubcoreMesh(core_axis_name="c", subcore_axis_name="s")
    run = pl.kernel(
        _body,
        out_type=jax.ShapeDtypeStruct((N + 1, N), jnp.float32),
        mesh=mesh,
        compiler_params=pltpu.CompilerParams(needs_layout_passes=False),
        scratch_types=[
            pltpu.VMEM((C, N), jnp.float32),
            pltpu.VMEM((C, N), jnp.float32),
            pltpu.VMEM((C, N), jnp.float32),
            pltpu.VMEM((ROWS_PER_W + L, ), jnp.float32),
            pltpu.VMEM((1, N), jnp.float32),
            pltpu.SemaphoreType.DMA,
            pltpu.SemaphoreType.DMA,
            pltpu.SemaphoreType.DMA,
            pltpu.SemaphoreType.DMA,
        ],
    )
    return run(x, eps)


# speedup vs baseline: 1.0118x; 1.0118x over previous
import jax
import jax.numpy as jnp
from jax import lax
from jax.experimental import pallas as pl
from jax.experimental.pallas import tpu as pltpu
from jax.experimental.pallas import tpu_sc as plsc

N = 4096
NC = 2
NS = 16
NW = NC * NS
ROWS_PER_W = N // NW
C = 8
NCHUNK = ROWS_PER_W // C
L = 16


def _body(x_hbm, eps_hbm, out_hbm, buf0, buf1, buf2, epsv, xrow,
          sem0, sem1, sem2, esem, xsem):
    wid = lax.axis_index("s") * NC + lax.axis_index("c")
    base = wid * ROWS_PER_W

    src_off = jnp.where(wid == 0, 0, base - C)
    dst_off = jnp.where(wid == 0, C, 0)
    eps_cp = pltpu.make_async_copy(
        eps_hbm.at[pl.ds(src_off, ROWS_PER_W + L - C)],
        epsv.at[pl.ds(dst_off, ROWS_PER_W + L - C)],
        esem,
    )
    eps_cp.start()

    x_cp = pltpu.make_async_copy(x_hbm, xrow, xsem)

    @pl.when(wid == 0)
    def _():
        x_cp.start()

    zeros = jnp.zeros((L,), jnp.float32)

    def _zero_buf(buf):
        for j in range(C):
            def trip(i, _):
                b0 = i * (8 * L)
                for u in range(8):
                    buf[j, pl.ds(b0 + u * L, L)] = zeros
                return 0
            lax.fori_loop(0, N // (8 * L), trip, 0)

    bufs = (buf0, buf1, buf2)
    sems = (sem0, sem1, sem2)
    NB = len(bufs)
    pending = [None, None, None]
    need_clear = [False, False, False]

    lane = lax.iota(jnp.int32, L)
    lane_ok = (lane >= C - 1) & (lane < L - 1)

    def diag_pos(k):
        col = base - C + 8 * k + lane
        m = lane_ok & (col >= 0)
        return m, jnp.where(m, lane - (C - 1), 0), jnp.where(m, col, 0)

    def run_chunk(k):
        b = k % NB
        if pending[b] is not None:
            pending[b].wait()
            pending[b] = None
        if need_clear[b]:
            om, orow, ocol = diag_pos(k - NB)
            plsc.store_scatter(bufs[b], [orow, ocol], zeros, mask=om)
        m, row, col = diag_pos(k)
        ev = epsv[pl.ds(8 * k, L)]
        plsc.store_scatter(bufs[b], [row, col], ev, mask=m)
        pending[b] = pltpu.async_copy(
            bufs[b],
            out_hbm.at[pl.ds(base + 8 * k, C)],
            sems[b],
        )
        need_clear[b] = True

    _zero_buf(buf0)
    eps_cp.wait()
    run_chunk(0)
    _zero_buf(buf1)
    run_chunk(1)
    _zero_buf(buf2)
    run_chunk(2)

    @pl.when(wid == NW - 1)
    def _():
        def ztrip(i, _):
            b0 = i * (8 * L)
            for u in range(8):
                xrow[0, pl.ds(b0 + u * L, L)] = zeros
            return 0
        lax.fori_loop(0, N // (8 * L), ztrip, 0)
        one = lane == C - 1
        ev = epsv[pl.ds(8 * NCHUNK, L)]
        plsc.store_scatter(xrow, [jnp.zeros((L,), jnp.int32),
                                  jnp.where(one, N - 1, 0)], ev, mask=one)
        pltpu.make_async_copy(xrow, out_hbm.at[pl.ds(N, 1)], xsem).start()

    run_chunk(3)

    @pl.when(wid == 0)
    def _():
        x_cp.wait()
        pltpu.make_async_copy(xrow, out_hbm.at[pl.ds(0, 1)], xsem).start()

    for k in range(4, NCHUNK):
        run_chunk(k)
    for p in pending:
        if p is not None:
            p.wait()

    @pl.when(wid == 0)
    def _():
        pltpu.make_async_copy(xrow, out_hbm.at[pl.ds(0, 1)], xsem).wait()

    @pl.when(wid == NW - 1)
    def _():
        pltpu.make_async_copy(xrow, out_hbm.at[pl.ds(N, 1)], xsem).wait()


@jax.jit
def kernel(x, eps):
    mesh = plsc.VectorSubcoreMesh(core_axis_name="c", subcore_axis_name="s")
    run = pl.kernel(
        _body,
        out_type=jax.ShapeDtypeStruct((N + 1, N), jnp.float32),
        mesh=mesh,
        compiler_params=pltpu.CompilerParams(needs_layout_passes=False),
        scratch_types=[
            pltpu.VMEM((C, N), jnp.float32),
            pltpu.VMEM((C, N), jnp.float32),
            pltpu.VMEM((C, N), jnp.float32),
            pltpu.VMEM((ROWS_PER_W + L, ), jnp.float32),
            pltpu.VMEM((1, N), jnp.float32),
            pltpu.SemaphoreType.DMA,
            pltpu.SemaphoreType.DMA,
            pltpu.SemaphoreType.DMA,
            pltpu.SemaphoreType.DMA,
            pltpu.SemaphoreType.DMA,
        ],
    )
    return run(x, eps)
